# Initial kernel scaffold; baseline (speedup 1.0000x reference)
import jax
import jax.numpy as jnp
from jax.experimental import pallas as pl


def kernel(x, y, edge_index, etypes, W_rel, loop_w, bias, gate_w, gate_b):
    raise NotImplementedError("write your pallas kernel here")


# speedup vs baseline: 1.0087x; 1.0087x over previous
import jax
import jax.numpy as jnp
from jax.experimental import pallas as pl
from jax.experimental.pallas import tpu as pltpu

_N = 100000
_R = 27
_K = 512
_IN = 2
_TN = 800
_GRID = _N // _TN


def _pool_body(feat_ref, wc_ref, b_ref, gw_ref, out_ref, m_ref, s_ref, v_ref):
    i = pl.program_id(0)

    @pl.when(i == 0)
    def _init():
        m_ref[0] = -jnp.inf
        s_ref[0] = 0.0
        v_ref[...] = jnp.zeros_like(v_ref)

    h = jnp.dot(feat_ref[...], wc_ref[...], preferred_element_type=jnp.float32)
    h = jnp.maximum(h + b_ref[...], 0.0)
    g = jnp.sum(h * gw_ref[...], axis=1, keepdims=True)
    gm = jnp.max(g)
    m_old = m_ref[0]
    m_new = jnp.maximum(m_old, gm)
    scale = jnp.exp(m_old - m_new)
    p = jnp.exp(g - m_new)
    s_ref[0] = s_ref[0] * scale + jnp.sum(p)
    v_ref[...] = v_ref[...] * scale + jnp.sum(p * h, axis=0, keepdims=True)
    m_ref[0] = m_new

    @pl.when(i == _GRID - 1)
    def _fin():
        out_ref[...] = v_ref[...] / s_ref[0]


def _attention_pool(feats, wc, bias, gate_w):
    return pl.pallas_call(
        _pool_body,
        grid=(_GRID,),
        in_specs=[
            pl.BlockSpec((_TN, 64), lambda i: (i, 0)),
            pl.BlockSpec((64, _K), lambda i: (0, 0)),
            pl.BlockSpec((1, _K), lambda i: (0, 0)),
            pl.BlockSpec((1, _K), lambda i: (0, 0)),
        ],
        out_specs=pl.BlockSpec((1, _K), lambda i: (0, 0)),
        out_shape=jax.ShapeDtypeStruct((1, _K), jnp.float32),
        scratch_shapes=[
            pltpu.SMEM((1,), jnp.float32),
            pltpu.SMEM((1,), jnp.float32),
            pltpu.VMEM((1, _K), jnp.float32),
        ],
    )(feats, wc, bias, gate_w)


def kernel(x, y, edge_index, etypes, W_rel, loop_w, bias, gate_w, gate_b):
    x2 = x.reshape(x.shape[1], -1)
    src = edge_index[0]
    dst = edge_index[1]
    seg = dst * _R + etypes
    acc = jax.ops.segment_sum(x2[src], seg, num_segments=_N * _R)
    acc = acc.reshape(_N, _R * _IN)
    feats = jnp.concatenate(
        [acc, x2, jnp.zeros((_N, 8), jnp.float32)], axis=1)
    wc = jnp.concatenate(
        [W_rel.reshape(_R * _IN, _K), loop_w, jnp.zeros((8, _K), jnp.float32)],
        axis=0)
    return _attention_pool(feats, wc, bias.reshape(1, _K), gate_w.reshape(1, _K))
